# Initial kernel scaffold; baseline (speedup 1.0000x reference)
import jax
import jax.numpy as jnp
from jax.experimental import pallas as pl


def kernel(x, weight, bias):
    raise NotImplementedError("write your pallas kernel here")


# speedup vs baseline: 19.8850x; 19.8850x over previous
import jax
import jax.numpy as jnp
from jax.experimental import pallas as pl
from jax.experimental.pallas import tpu as pltpu

_VMEM_LIMIT = 48 * 1024 * 1024


def _fused_ds_kernel(p_ref, w_ref, b_ref, yc_ref, yp_ref):
    _, hp, _, c4 = p_ref.shape
    ho = hp - 1
    s = yc_ref.shape[1]
    wo = s // ho
    c = c4 // 4

    s00 = p_ref[0, 1:1 + ho, 1:1 + wo, :].reshape(s, 4 * c)
    acc = jnp.dot(s00, w_ref[0:4 * c], preferred_element_type=jnp.float32)
    t = p_ref[0, 0:ho, 1:1 + wo, 2 * c:4 * c].reshape(s, 2 * c)
    acc += jnp.dot(t, w_ref[4 * c:6 * c], preferred_element_type=jnp.float32)
    t = p_ref[0, 1:1 + ho, 0:wo, c:2 * c].reshape(s, c)
    acc += jnp.dot(t, w_ref[6 * c:7 * c], preferred_element_type=jnp.float32)
    t = p_ref[0, 1:1 + ho, 0:wo, 3 * c:4 * c].reshape(s, c)
    acc += jnp.dot(t, w_ref[7 * c:8 * c], preferred_element_type=jnp.float32)
    t = p_ref[0, 0:ho, 0:wo, 3 * c:4 * c].reshape(s, c)
    acc += jnp.dot(t, w_ref[8 * c:9 * c], preferred_element_type=jnp.float32)
    yc_ref[0] = acc + b_ref[...]

    ps = (s00[:, 0:c].astype(jnp.float32)
          + s00[:, c:2 * c].astype(jnp.float32)
          + s00[:, 2 * c:3 * c].astype(jnp.float32)
          + s00[:, 3 * c:4 * c].astype(jnp.float32))
    yp_ref[0] = 0.25 * ps


def kernel(x, weight, bias):
    n, c, h, w = x.shape
    cout = weight.shape[0]
    ho, wo = h // 2, w // 2
    s = ho * wo

    p = x.reshape(n, c, ho, 2, wo, 2)
    p = jnp.transpose(p, (0, 2, 4, 3, 5, 1)).reshape(n, ho, wo, 4 * c)
    p = p.astype(jnp.bfloat16)
    wop = wo + 1 + (-(wo + 1) % 8)
    p = jnp.pad(p, ((0, 0), (1, 0), (1, wop - wo - 1), (0, 0)))

    wt = jnp.transpose(weight, (2, 3, 1, 0)).astype(jnp.bfloat16)
    wm = jnp.concatenate(
        [wt[1, 1], wt[1, 2], wt[2, 1], wt[2, 2],
         wt[0, 1], wt[0, 2],
         wt[1, 0], wt[2, 0],
         wt[0, 0]], axis=0)
    b2 = bias.reshape(1, cout).astype(jnp.float32)

    yc, yp = pl.pallas_call(
        _fused_ds_kernel,
        out_shape=(jax.ShapeDtypeStruct((n, s, cout), jnp.float32),
                   jax.ShapeDtypeStruct((n, s, cout), jnp.float32)),
        grid=(n,),
        in_specs=[
            pl.BlockSpec((1, ho + 1, wop, 4 * c), lambda i: (i, 0, 0, 0)),
            pl.BlockSpec((9 * c, cout), lambda i: (0, 0)),
            pl.BlockSpec((1, cout), lambda i: (0, 0)),
        ],
        out_specs=(pl.BlockSpec((1, s, cout), lambda i: (i, 0, 0)),
                   pl.BlockSpec((1, s, cout), lambda i: (i, 0, 0))),
        compiler_params=pltpu.CompilerParams(
            dimension_semantics=("parallel",),
            vmem_limit_bytes=_VMEM_LIMIT,
        ),
        cost_estimate=pl.CostEstimate(
            flops=2 * n * s * 9 * c * cout,
            transcendentals=0,
            bytes_accessed=(n * (ho + 1) * wop * 4 * c * 2
                            + 9 * c * cout * 2 + 2 * n * s * cout * 4),
        ),
    )(p, wm, b2)

    yc = jnp.transpose(yc, (0, 2, 1)).reshape(n, cout, ho, wo)
    yp = jnp.transpose(yp, (0, 2, 1)).reshape(n, c, ho, wo)
    return yc, yp


# speedup vs baseline: 29.4708x; 1.4821x over previous
import jax
import jax.numpy as jnp
from jax.experimental import pallas as pl
from jax.experimental.pallas import tpu as pltpu

_VMEM_LIMIT = 48 * 1024 * 1024


def kernel(x, weight, bias):
    n, c, h, w = x.shape
    cout = weight.shape[0]
    ho, wo = h // 2, w // 2
    s = ho * wo
    bf16 = jnp.bfloat16

    x3 = x.reshape(n, c, h * w)
    wt = jnp.transpose(weight, (2, 3, 1, 0))
    wm = wt.reshape(9 * c, cout).astype(bf16)
    eye = jnp.eye(c, dtype=bf16)
    e2 = jnp.concatenate([eye, 0.25 * eye], axis=0)
    b2 = bias.reshape(cout, 1).astype(jnp.float32)

    def body(x_ref, w_ref, e_ref, b_ref, yc_ref, yp_ref):
        xb = x_ref[0].astype(bf16)
        xt = jax.lax.dot_general(xb, e_ref[0:c],
                                 (((0,), (0,)), ((), ())),
                                 preferred_element_type=jnp.float32)
        x6 = xt.astype(bf16).reshape(h * wo, 2 * c).reshape(ho, 2, wo, 2 * c)

        base = [[x6[:, py, :, px * c:(px + 1) * c] for px in (0, 1)]
                for py in (0, 1)]
        zrow = jnp.zeros((1, wo, c), bf16)
        zcol = jnp.zeros((ho, 1, c), bf16)

        def tap_for(ky, kx):
            dy, py = ((-1, 1) if ky == 0 else (0, ky - 1))
            dx, px = ((-1, 1) if kx == 0 else (0, kx - 1))
            a = base[py][px]
            if dy:
                a = jnp.concatenate([zrow, a[0:ho - 1]], axis=0)
            if dx:
                a = jnp.concatenate([zcol, a[:, 0:wo - 1, :]], axis=1)
            return a

        acc = None
        pacc = None
        for ky in range(3):
            for kx in range(3):
                tap = tap_for(ky, kx).reshape(s, c)
                i = ky * 3 + kx
                d = jax.lax.dot_general(w_ref[i * c:(i + 1) * c], tap,
                                        (((0,), (1,)), ((), ())),
                                        preferred_element_type=jnp.float32)
                acc = d if acc is None else acc + d
                if ky >= 1 and kx >= 1:
                    p = jax.lax.dot_general(e_ref[c:2 * c], tap,
                                            (((0,), (1,)), ((), ())),
                                            preferred_element_type=jnp.float32)
                    pacc = p if pacc is None else pacc + p
        yc_ref[0] = acc + b_ref[...]
        yp_ref[0] = pacc

    yc, yp = pl.pallas_call(
        body,
        out_shape=(jax.ShapeDtypeStruct((n, cout, s), jnp.float32),
                   jax.ShapeDtypeStruct((n, c, s), jnp.float32)),
        grid=(n,),
        in_specs=[
            pl.BlockSpec((1, c, h * w), lambda i: (i, 0, 0)),
            pl.BlockSpec((9 * c, cout), lambda i: (0, 0)),
            pl.BlockSpec((2 * c, c), lambda i: (0, 0)),
            pl.BlockSpec((cout, 1), lambda i: (0, 0)),
        ],
        out_specs=(pl.BlockSpec((1, cout, s), lambda i: (i, 0, 0)),
                   pl.BlockSpec((1, c, s), lambda i: (i, 0, 0))),
        compiler_params=pltpu.CompilerParams(
            dimension_semantics=("parallel",),
            vmem_limit_bytes=_VMEM_LIMIT,
        ),
        cost_estimate=pl.CostEstimate(
            flops=2 * n * s * (9 + 4) * c * cout + 2 * n * h * w * c * c,
            transcendentals=0,
            bytes_accessed=(n * c * h * w * 4 + 9 * c * cout * 2
                            + n * s * (c + cout) * 4),
        ),
    )(x3, wm, e2, b2)

    return yc.reshape(n, cout, ho, wo), yp.reshape(n, c, ho, wo)
